# baseline (speedup 1.0000x reference)
import functools

import jax
import jax.numpy as jnp
from jax import lax
from jax.experimental import pallas as pl
from jax.experimental.pallas import tpu as pltpu
from jax.experimental.pallas import tpu_sc as plsc

NC = 2
NS = 16
NW = NC * NS
CHUNK = 128
NROW = 2
NIDX = 4


GSPLIT = 2


def _gather_split(table_h, idx_ref, rows_ref, sem):
    hw = CHUNK // GSPLIT
    for h in range(GSPLIT):
        pltpu.async_copy(table_h.at[idx_ref.at[pl.ds(h * hw, hw)]],
                         rows_ref.at[pl.ds(h * hw, hw)], sem)


def _gather_split_wait(table_h, idx_ref, rows_ref, sem):
    hw = CHUNK // GSPLIT
    for h in range(GSPLIT):
        pltpu.make_async_copy(table_h.at[idx_ref.at[pl.ds(h * hw, hw)]],
                              rows_ref.at[pl.ds(h * hw, hw)], sem).wait()


def _seg_sum_sc(table, g2, s2, ew2):
    n, f = table.shape
    tot_chunks = g2.shape[0]
    nsl = f // 16
    rpt = (n // NS) // 8 * 8
    rem = n - rpt * NS
    nfull = rpt // CHUNK
    ztail = rpt - nfull * CHUNK
    assert rem % 8 == 0 and rem <= CHUNK and ztail % 8 == 0
    base_ck = (tot_chunks // NW) // NIDX * NIDX
    nhi = (tot_chunks - base_ck * NW) // NIDX
    wlo = NW - nhi
    assert base_ck * NW + nhi * NIDX == tot_chunks

    mesh = plsc.VectorSubcoreMesh(core_axis_name="c", subcore_axis_name="s")

    @functools.partial(
        pl.kernel,
        out_type=jax.ShapeDtypeStruct((NC, n, f), jnp.float32),
        mesh=mesh,
        scratch_types=[
            [pltpu.VMEM((CHUNK,), jnp.int32)] * NIDX,
            [pltpu.VMEM((CHUNK,), jnp.int32)] * NIDX,
            [pltpu.VMEM((CHUNK,), jnp.float32)] * NIDX,
            [pltpu.VMEM((CHUNK, f), jnp.float32)] * NROW,
            pltpu.VMEM_SHARED((n, f), jnp.float32),
            [pltpu.SemaphoreType.DMA] * NIDX,
            [pltpu.SemaphoreType.DMA] * NROW,
            [pltpu.SemaphoreType.DMA] * NROW,
        ],
    )
    def ksc(table_h, g_h, s_h, ew_h, out_h, g_b, s_b, ew_b, rows, acc,
            isem, gsem, ssem):
        c_ax = lax.axis_index("c")
        s_ax = lax.axis_index("s")
        wid = c_ax * NS + s_ax
        nck = base_ck + jnp.where(wid >= wlo, NIDX, 0)
        cbase = wid * base_ck + jnp.maximum(0, wid - wlo) * NIDX
        zero16 = jnp.zeros((16,), jnp.float32)

        def idx_load(rel_c, st):
            gc = cbase + rel_c
            pltpu.async_copy(g_h.at[gc], g_b[st], isem[st])
            pltpu.async_copy(s_h.at[gc], s_b[st], isem[st])
            pltpu.async_copy(ew_h.at[gc], ew_b[st], isem[st])

        def idx_wait(rel_c, st):
            gc = cbase + rel_c
            pltpu.make_async_copy(g_h.at[gc], g_b[st], isem[st]).wait()
            pltpu.make_async_copy(s_h.at[gc], s_b[st], isem[st]).wait()
            pltpu.make_async_copy(ew_h.at[gc], ew_b[st], isem[st]).wait()

        for st in range(3):
            idx_load(st, st)

        def zrow(r, carry):
            for j in range(nsl):
                rows[0][r, pl.ds(j * 16, 16)] = zero16
            return carry

        lax.fori_loop(0, CHUNK, zrow, 0)
        row0 = pl.multiple_of(s_ax * rpt, 8)
        for zi in range(nfull):
            pltpu.sync_copy(rows[0], acc.at[pl.ds(row0 + zi * CHUNK, CHUNK)])
        if ztail:
            pltpu.sync_copy(rows[0].at[pl.ds(0, ztail)],
                            acc.at[pl.ds(row0 + nfull * CHUNK, ztail)])
        if rem:
            @pl.when(s_ax == 0)
            def _():
                pltpu.sync_copy(rows[0].at[pl.ds(0, rem)],
                                acc.at[pl.ds(NS * rpt, rem)])
        plsc.subcore_barrier()

        idx_wait(0, 0)
        _gather_split(table_h, g_b[0], rows[0], gsem[0])

        def phase(i, b):
            c = i * NIDX + b
            br = b % NROW
            b1 = (b + 1) % NROW
            st1 = (b + 1) % NIDX
            st3 = (b + 3) % NIDX
            _gather_split_wait(table_h, g_b[b], rows[br], gsem[br])
            for g in range(CHUNK // 16):
                wv = ew_b[b][pl.ds(g * 16, 16)]

                def lane_body(l, cc2, _wv=wv, _br=br, _g=g):
                    k = _g * 16 + l
                    w = lax.gather(
                        _wv, jnp.full((16, 1), l, jnp.int32),
                        lax.GatherDimensionNumbers(
                            offset_dims=(), collapsed_slice_dims=(0,),
                            start_index_map=(0,)),
                        slice_sizes=(1,),
                        mode=lax.GatherScatterMode.PROMISE_IN_BOUNDS)
                    for j in range(nsl):
                        rows[_br][k, pl.ds(j * 16, 16)] = (
                            rows[_br][k, pl.ds(j * 16, 16)] * w)
                    return cc2

                lax.fori_loop(0, 16, lane_body, 0, unroll=2)
            pltpu.async_copy(rows[br], acc.at[s_b[b]], ssem[br], add=True)

            @pl.when(c + 1 < nck)
            def _():
                @pl.when(c >= 1)
                def _():
                    pltpu.make_async_copy(rows[b1], acc.at[s_b[st3]],
                                          ssem[b1]).wait()

                @pl.when(c + 3 < nck)
                def _():
                    idx_load(c + 3, st3)
                idx_wait(c + 1, st1)
                _gather_split(table_h, g_b[st1], rows[b1], gsem[b1])

        def outer(i, carry):
            for b in range(NIDX):
                phase(i, b)
            return carry

        lax.fori_loop(0, nck // NIDX, outer, 0)

        for br in range(NROW):
            pltpu.make_async_copy(rows[br], acc.at[s_b[0]], ssem[br]).wait()

        plsc.subcore_barrier()
        pltpu.sync_copy(acc.at[pl.ds(row0, rpt)],
                        out_h.at[c_ax, pl.ds(row0, rpt)])
        if rem:
            @pl.when(s_ax == 0)
            def _():
                pltpu.sync_copy(acc.at[pl.ds(NS * rpt, rem)],
                                out_h.at[c_ax, pl.ds(NS * rpt, rem)])

    return ksc(table, g2, s2, ew2)


def _tc_layer1(P, x, wn, ws, b):
    n = x.shape[0]
    bn = 1000

    def body(p_ref, x_ref, wn_ref, ws_ref, b_ref, o_ref):
        agg = p_ref[0] + p_ref[1]
        h = (jnp.dot(agg, wn_ref[...], preferred_element_type=jnp.float32)
             + jnp.dot(x_ref[...], ws_ref[...], preferred_element_type=jnp.float32)
             + b_ref[...])
        o_ref[...] = jnp.maximum(h, 0.0)

    return pl.pallas_call(
        body,
        grid=(n // bn,),
        in_specs=[
            pl.BlockSpec((2, bn, 128), lambda i: (0, i, 0)),
            pl.BlockSpec((bn, 128), lambda i: (i, 0)),
            pl.BlockSpec((128, 128), lambda i: (0, 0)),
            pl.BlockSpec((128, 128), lambda i: (0, 0)),
            pl.BlockSpec((1, 128), lambda i: (0, 0)),
        ],
        out_specs=pl.BlockSpec((bn, 128), lambda i: (i, 0)),
        out_shape=jax.ShapeDtypeStruct((n, 128), jnp.float32),
    )(P, x, wn, ws, b)


def _tc_layer2(Q, H, wn, ws, b):
    n = H.shape[0]
    bn = 1000

    def body(q_ref, h_ref, wn_ref, ws_ref, b_ref, s_ref, lx_ref):
        t = (jnp.dot(q_ref[0] + q_ref[1], wn_ref[...], preferred_element_type=jnp.float32)
             + jnp.dot(h_ref[...], ws_ref[...], preferred_element_type=jnp.float32)
             + b_ref[...])
        sblk = t[:, :64]
        s_ref[...] = sblk
        m = jnp.max(sblk, axis=1, keepdims=True)
        sh = sblk - m
        ls = sh - jnp.log(jnp.sum(jnp.exp(sh), axis=1, keepdims=True))
        lx_ref[...] = jnp.concatenate([ls, t[:, 64:]], axis=1)

    return pl.pallas_call(
        body,
        grid=(n // bn,),
        in_specs=[
            pl.BlockSpec((2, bn, 128), lambda i: (0, i, 0)),
            pl.BlockSpec((bn, 128), lambda i: (i, 0)),
            pl.BlockSpec((128, 128), lambda i: (0, 0)),
            pl.BlockSpec((128, 128), lambda i: (0, 0)),
            pl.BlockSpec((1, 128), lambda i: (0, 0)),
        ],
        out_specs=[
            pl.BlockSpec((bn, 64), lambda i: (i, 0)),
            pl.BlockSpec((bn, 128), lambda i: (i, 0)),
        ],
        out_shape=[
            jax.ShapeDtypeStruct((n, 64), jnp.float32),
            jax.ShapeDtypeStruct((n, 128), jnp.float32),
        ],
    )(Q, H, wn, ws, b)


def _tc_final(LX, R, wn1, ws1, b1, wn2, ws2, b2, l1w, l1b, l2w, l2b):
    n = LX.shape[0]
    bn = 1000
    nb = n // bn
    cdim = (((0,), (0,)), ((), ()))

    def body(lx_ref, r_ref, wn1_ref, ws1_ref, b1_ref, wn2_ref, ws2_ref,
             b2_ref, l1w_ref, l1b_ref, l2w_ref, l2b_ref, out_ref, x1_ref,
             a2_acc, xp_acc):
        i = pl.program_id(0)

        @pl.when(i == 0)
        def _():
            a2_acc[...] = jnp.zeros_like(a2_acc)
            xp_acc[...] = jnp.zeros_like(xp_acc)

        lx = lx_ref[...]
        sb = lx[:, :64]
        asb = r_ref[0, :, :64] + r_ref[1, :, :64]
        a2_acc[...] += lax.dot_general(sb, asb, cdim,
                                       preferred_element_type=jnp.float32)
        xp_acc[...] += lax.dot_general(sb, lx[:, 64:], cdim,
                                       preferred_element_type=jnp.float32)

        @pl.when(i == nb - 1)
        def _():
            A2 = a2_acc[...]
            xp = xp_acc[...]
            x1_ref[...] = xp
            h = jnp.maximum(
                jnp.dot(jnp.dot(A2, xp, preferred_element_type=jnp.float32),
                        wn1_ref[...], preferred_element_type=jnp.float32)
                + jnp.dot(xp, ws1_ref[...], preferred_element_type=jnp.float32)
                + b1_ref[...], 0.0)
            h = (jnp.dot(jnp.dot(A2, h, preferred_element_type=jnp.float32),
                         wn2_ref[...], preferred_element_type=jnp.float32)
                 + jnp.dot(h, ws2_ref[...], preferred_element_type=jnp.float32)
                 + b2_ref[...])
            xm = jnp.mean(h, axis=0, keepdims=True)
            z = jnp.dot(xm, l1w_ref[...], preferred_element_type=jnp.float32) + l1b_ref[...]
            z = jnp.where(z >= 0, z, 0.01 * z)
            z = jnp.dot(z, l2w_ref[...], preferred_element_type=jnp.float32) + l2b_ref[...]
            m = jnp.max(z, axis=1, keepdims=True)
            zs = z - m
            out_ref[...] = zs - jnp.log(jnp.sum(jnp.exp(zs), axis=1, keepdims=True))

    full = lambda shape: pl.BlockSpec(shape, lambda i: tuple(0 for _ in shape))
    return pl.pallas_call(
        body,
        grid=(nb,),
        in_specs=[
            pl.BlockSpec((bn, 128), lambda i: (i, 0)),
            pl.BlockSpec((2, bn, 128), lambda i: (0, i, 0)),
            full((64, 64)), full((64, 64)), full((1, 64)),
            full((64, 64)), full((64, 64)), full((1, 64)),
            full((64, 32)), full((1, 32)), full((32, 32)), full((1, 32)),
        ],
        out_specs=[
            pl.BlockSpec((1, 32), lambda i: (0, 0)),
            pl.BlockSpec((64, 64), lambda i: (0, 0)),
        ],
        out_shape=[
            jax.ShapeDtypeStruct((1, 32), jnp.float32),
            jax.ShapeDtypeStruct((64, 64), jnp.float32),
        ],
        scratch_shapes=[
            pltpu.VMEM((64, 64), jnp.float32),
            pltpu.VMEM((64, 64), jnp.float32),
        ],
    )(LX, R, wn1, ws1, b1, wn2, ws2, b2, l1w, l1b, l2w, l2b)


def kernel(x, edge_index, edge_weight, params):
    src = edge_index[0]
    dst = edge_index[1]
    (wn_p1, ws_p1, b_p1), (wn_p2, ws_p2, b_p2) = params['pool']
    (wn_e1, ws_e1, b_e1), (wn_e2, ws_e2, b_e2) = params['embed']
    wn1 = jnp.concatenate([wn_p1, wn_e1], axis=1)
    ws1 = jnp.concatenate([ws_p1, ws_e1], axis=1)
    b1 = jnp.concatenate([b_p1, b_e1])[None, :]
    z64 = jnp.zeros((64, 64), jnp.float32)
    wn2 = jnp.block([[wn_p2, z64], [z64, wn_e2]])
    ws2 = jnp.block([[ws_p2, z64], [z64, ws_e2]])
    b2 = jnp.concatenate([b_p2, b_e2])[None, :]

    e = edge_weight.shape[0]
    src2 = src.reshape(e // CHUNK, CHUNK)
    dst2 = dst.reshape(e // CHUNK, CHUNK)
    ew2 = edge_weight.reshape(e // CHUNK, CHUNK)

    P = _seg_sum_sc(x, src2, dst2, ew2)
    H = _tc_layer1(P, x, wn1, ws1, b1)
    Q = _seg_sum_sc(H, src2, dst2, ew2)
    s1, LX = _tc_layer2(Q, H, wn2, ws2, b2)
    R = _seg_sum_sc(LX, dst2, src2, ew2)

    (g2wn1, g2ws1, g2b1), (g2wn2, g2ws2, g2b2) = params['gnn2']
    out, x1 = _tc_final(LX, R,
                        g2wn1, g2ws1, g2b1[None, :],
                        g2wn2, g2ws2, g2b2[None, :],
                        params['lin1_w'], params['lin1_b'][None, :],
                        params['lin2_w'], params['lin2_b'][None, :])
    return (out, s1, x1)


# speedup vs baseline: 1.2669x; 1.2669x over previous
import functools

import jax
import jax.numpy as jnp
from jax import lax
from jax.experimental import pallas as pl
from jax.experimental.pallas import tpu as pltpu
from jax.experimental.pallas import tpu_sc as plsc

NC = 2
NS = 16
NW = NC * NS
CHUNK = 128
NROW = 2
NIDX = 4


GSPLIT = 2


def _gather_split(table_h, idx_ref, rows_ref, sem):
    hw = CHUNK // GSPLIT
    for h in range(GSPLIT):
        pltpu.async_copy(table_h.at[idx_ref.at[pl.ds(h * hw, hw)]],
                         rows_ref.at[pl.ds(h * hw, hw)], sem)


def _gather_split_wait(table_h, idx_ref, rows_ref, sem):
    hw = CHUNK // GSPLIT
    for h in range(GSPLIT):
        pltpu.make_async_copy(table_h.at[idx_ref.at[pl.ds(h * hw, hw)]],
                              rows_ref.at[pl.ds(h * hw, hw)], sem).wait()


def _seg_sum_sc(table, g2, s2, ew2):
    n, f = table.shape
    tot_chunks = g2.shape[0]
    nsl = f // 16
    rpt = (n // NS) // 8 * 8
    rem = n - rpt * NS
    nfull = rpt // CHUNK
    ztail = rpt - nfull * CHUNK
    assert rem % 8 == 0 and rem <= CHUNK and ztail % 8 == 0
    base_ck = (tot_chunks // NW) // NIDX * NIDX
    nhi = (tot_chunks - base_ck * NW) // NIDX
    wlo = NW - nhi
    assert base_ck * NW + nhi * NIDX == tot_chunks

    mesh = plsc.VectorSubcoreMesh(core_axis_name="c", subcore_axis_name="s")

    @functools.partial(
        pl.kernel,
        out_type=jax.ShapeDtypeStruct((NC, n, f), jnp.float32),
        mesh=mesh,
        scratch_types=[
            [pltpu.VMEM((CHUNK,), jnp.int32)] * NIDX,
            [pltpu.VMEM((CHUNK,), jnp.int32)] * NIDX,
            [pltpu.VMEM((CHUNK,), jnp.float32)] * NIDX,
            [pltpu.VMEM((CHUNK, f), jnp.float32)] * NROW,
            pltpu.VMEM_SHARED((n, f), jnp.float32),
            [pltpu.SemaphoreType.DMA] * NIDX,
            [pltpu.SemaphoreType.DMA] * NROW,
            [pltpu.SemaphoreType.DMA] * NROW,
        ],
    )
    def ksc(table_h, g_h, s_h, ew_h, out_h, g_b, s_b, ew_b, rows, acc,
            isem, gsem, ssem):
        c_ax = lax.axis_index("c")
        s_ax = lax.axis_index("s")
        wid = c_ax * NS + s_ax
        nck = base_ck + jnp.where(wid >= wlo, NIDX, 0)
        cbase = wid * base_ck + jnp.maximum(0, wid - wlo) * NIDX
        zero16 = jnp.zeros((16,), jnp.float32)

        def idx_load(rel_c, st):
            gc = cbase + rel_c
            pltpu.async_copy(g_h.at[gc], g_b[st], isem[st])
            pltpu.async_copy(s_h.at[gc], s_b[st], isem[st])
            pltpu.async_copy(ew_h.at[gc], ew_b[st], isem[st])

        def idx_wait(rel_c, st):
            gc = cbase + rel_c
            pltpu.make_async_copy(g_h.at[gc], g_b[st], isem[st]).wait()
            pltpu.make_async_copy(s_h.at[gc], s_b[st], isem[st]).wait()
            pltpu.make_async_copy(ew_h.at[gc], ew_b[st], isem[st]).wait()

        for st in range(3):
            idx_load(st, st)

        def zrow(r, carry):
            for j in range(nsl):
                rows[0][r, pl.ds(j * 16, 16)] = zero16
            return carry

        lax.fori_loop(0, CHUNK, zrow, 0)
        row0 = pl.multiple_of(s_ax * rpt, 8)
        for zi in range(nfull):
            pltpu.sync_copy(rows[0], acc.at[pl.ds(row0 + zi * CHUNK, CHUNK)])
        if ztail:
            pltpu.sync_copy(rows[0].at[pl.ds(0, ztail)],
                            acc.at[pl.ds(row0 + nfull * CHUNK, ztail)])
        if rem:
            @pl.when(s_ax == 0)
            def _():
                pltpu.sync_copy(rows[0].at[pl.ds(0, rem)],
                                acc.at[pl.ds(NS * rpt, rem)])
        plsc.subcore_barrier()

        idx_wait(0, 0)
        _gather_split(table_h, g_b[0], rows[0], gsem[0])

        def phase(i, b):
            c = i * NIDX + b
            br = b % NROW
            b1 = (b + 1) % NROW
            st1 = (b + 1) % NIDX
            st3 = (b + 3) % NIDX
            _gather_split_wait(table_h, g_b[b], rows[br], gsem[br])

            @pl.when(c + 1 < nck)
            def _():
                @pl.when(c >= 1)
                def _():
                    pltpu.make_async_copy(rows[b1], acc.at[s_b[st3]],
                                          ssem[b1]).wait()

                @pl.when(c + 3 < nck)
                def _():
                    idx_load(c + 3, st3)
                idx_wait(c + 1, st1)
                _gather_split(table_h, g_b[st1], rows[b1], gsem[b1])

            for g in range(CHUNK // 16):
                wv = ew_b[b][pl.ds(g * 16, 16)]

                def lane_body(l, cc2, _wv=wv, _br=br, _g=g):
                    k = _g * 16 + l
                    w = lax.gather(
                        _wv, jnp.full((16, 1), l, jnp.int32),
                        lax.GatherDimensionNumbers(
                            offset_dims=(), collapsed_slice_dims=(0,),
                            start_index_map=(0,)),
                        slice_sizes=(1,),
                        mode=lax.GatherScatterMode.PROMISE_IN_BOUNDS)
                    for j in range(nsl):
                        rows[_br][k, pl.ds(j * 16, 16)] = (
                            rows[_br][k, pl.ds(j * 16, 16)] * w)
                    return cc2

                lax.fori_loop(0, 16, lane_body, 0, unroll=2)
            pltpu.async_copy(rows[br], acc.at[s_b[b]], ssem[br], add=True)

        def outer(i, carry):
            for b in range(NIDX):
                phase(i, b)
            return carry

        lax.fori_loop(0, nck // NIDX, outer, 0)

        for br in range(NROW):
            pltpu.make_async_copy(rows[br], acc.at[s_b[0]], ssem[br]).wait()

        plsc.subcore_barrier()
        pltpu.sync_copy(acc.at[pl.ds(row0, rpt)],
                        out_h.at[c_ax, pl.ds(row0, rpt)])
        if rem:
            @pl.when(s_ax == 0)
            def _():
                pltpu.sync_copy(acc.at[pl.ds(NS * rpt, rem)],
                                out_h.at[c_ax, pl.ds(NS * rpt, rem)])

    return ksc(table, g2, s2, ew2)


def _tc_layer1(P, x, wn, ws, b):
    n = x.shape[0]
    bn = 1000

    def body(p_ref, x_ref, wn_ref, ws_ref, b_ref, o_ref):
        agg = p_ref[0] + p_ref[1]
        h = (jnp.dot(agg, wn_ref[...], preferred_element_type=jnp.float32)
             + jnp.dot(x_ref[...], ws_ref[...], preferred_element_type=jnp.float32)
             + b_ref[...])
        o_ref[...] = jnp.maximum(h, 0.0)

    return pl.pallas_call(
        body,
        grid=(n // bn,),
        in_specs=[
            pl.BlockSpec((2, bn, 128), lambda i: (0, i, 0)),
            pl.BlockSpec((bn, 128), lambda i: (i, 0)),
            pl.BlockSpec((128, 128), lambda i: (0, 0)),
            pl.BlockSpec((128, 128), lambda i: (0, 0)),
            pl.BlockSpec((1, 128), lambda i: (0, 0)),
        ],
        out_specs=pl.BlockSpec((bn, 128), lambda i: (i, 0)),
        out_shape=jax.ShapeDtypeStruct((n, 128), jnp.float32),
    )(P, x, wn, ws, b)


def _tc_layer2(Q, H, wn, ws, b):
    n = H.shape[0]
    bn = 1000

    def body(q_ref, h_ref, wn_ref, ws_ref, b_ref, s_ref, lx_ref):
        t = (jnp.dot(q_ref[0] + q_ref[1], wn_ref[...], preferred_element_type=jnp.float32)
             + jnp.dot(h_ref[...], ws_ref[...], preferred_element_type=jnp.float32)
             + b_ref[...])
        sblk = t[:, :64]
        s_ref[...] = sblk
        m = jnp.max(sblk, axis=1, keepdims=True)
        sh = sblk - m
        ls = sh - jnp.log(jnp.sum(jnp.exp(sh), axis=1, keepdims=True))
        lx_ref[...] = jnp.concatenate([ls, t[:, 64:]], axis=1)

    return pl.pallas_call(
        body,
        grid=(n // bn,),
        in_specs=[
            pl.BlockSpec((2, bn, 128), lambda i: (0, i, 0)),
            pl.BlockSpec((bn, 128), lambda i: (i, 0)),
            pl.BlockSpec((128, 128), lambda i: (0, 0)),
            pl.BlockSpec((128, 128), lambda i: (0, 0)),
            pl.BlockSpec((1, 128), lambda i: (0, 0)),
        ],
        out_specs=[
            pl.BlockSpec((bn, 64), lambda i: (i, 0)),
            pl.BlockSpec((bn, 128), lambda i: (i, 0)),
        ],
        out_shape=[
            jax.ShapeDtypeStruct((n, 64), jnp.float32),
            jax.ShapeDtypeStruct((n, 128), jnp.float32),
        ],
    )(Q, H, wn, ws, b)


def _tc_final(LX, R, wn1, ws1, b1, wn2, ws2, b2, l1w, l1b, l2w, l2b):
    n = LX.shape[0]
    bn = 1000
    nb = n // bn
    cdim = (((0,), (0,)), ((), ()))

    def body(lx_ref, r_ref, wn1_ref, ws1_ref, b1_ref, wn2_ref, ws2_ref,
             b2_ref, l1w_ref, l1b_ref, l2w_ref, l2b_ref, out_ref, x1_ref,
             a2_acc, xp_acc):
        i = pl.program_id(0)

        @pl.when(i == 0)
        def _():
            a2_acc[...] = jnp.zeros_like(a2_acc)
            xp_acc[...] = jnp.zeros_like(xp_acc)

        lx = lx_ref[...]
        sb = lx[:, :64]
        asb = r_ref[0, :, :64] + r_ref[1, :, :64]
        a2_acc[...] += lax.dot_general(sb, asb, cdim,
                                       preferred_element_type=jnp.float32)
        xp_acc[...] += lax.dot_general(sb, lx[:, 64:], cdim,
                                       preferred_element_type=jnp.float32)

        @pl.when(i == nb - 1)
        def _():
            A2 = a2_acc[...]
            xp = xp_acc[...]
            x1_ref[...] = xp
            h = jnp.maximum(
                jnp.dot(jnp.dot(A2, xp, preferred_element_type=jnp.float32),
                        wn1_ref[...], preferred_element_type=jnp.float32)
                + jnp.dot(xp, ws1_ref[...], preferred_element_type=jnp.float32)
                + b1_ref[...], 0.0)
            h = (jnp.dot(jnp.dot(A2, h, preferred_element_type=jnp.float32),
                         wn2_ref[...], preferred_element_type=jnp.float32)
                 + jnp.dot(h, ws2_ref[...], preferred_element_type=jnp.float32)
                 + b2_ref[...])
            xm = jnp.mean(h, axis=0, keepdims=True)
            z = jnp.dot(xm, l1w_ref[...], preferred_element_type=jnp.float32) + l1b_ref[...]
            z = jnp.where(z >= 0, z, 0.01 * z)
            z = jnp.dot(z, l2w_ref[...], preferred_element_type=jnp.float32) + l2b_ref[...]
            m = jnp.max(z, axis=1, keepdims=True)
            zs = z - m
            out_ref[...] = zs - jnp.log(jnp.sum(jnp.exp(zs), axis=1, keepdims=True))

    full = lambda shape: pl.BlockSpec(shape, lambda i: tuple(0 for _ in shape))
    return pl.pallas_call(
        body,
        grid=(nb,),
        in_specs=[
            pl.BlockSpec((bn, 128), lambda i: (i, 0)),
            pl.BlockSpec((2, bn, 128), lambda i: (0, i, 0)),
            full((64, 64)), full((64, 64)), full((1, 64)),
            full((64, 64)), full((64, 64)), full((1, 64)),
            full((64, 32)), full((1, 32)), full((32, 32)), full((1, 32)),
        ],
        out_specs=[
            pl.BlockSpec((1, 32), lambda i: (0, 0)),
            pl.BlockSpec((64, 64), lambda i: (0, 0)),
        ],
        out_shape=[
            jax.ShapeDtypeStruct((1, 32), jnp.float32),
            jax.ShapeDtypeStruct((64, 64), jnp.float32),
        ],
        scratch_shapes=[
            pltpu.VMEM((64, 64), jnp.float32),
            pltpu.VMEM((64, 64), jnp.float32),
        ],
    )(LX, R, wn1, ws1, b1, wn2, ws2, b2, l1w, l1b, l2w, l2b)


def kernel(x, edge_index, edge_weight, params):
    src = edge_index[0]
    dst = edge_index[1]
    (wn_p1, ws_p1, b_p1), (wn_p2, ws_p2, b_p2) = params['pool']
    (wn_e1, ws_e1, b_e1), (wn_e2, ws_e2, b_e2) = params['embed']
    wn1 = jnp.concatenate([wn_p1, wn_e1], axis=1)
    ws1 = jnp.concatenate([ws_p1, ws_e1], axis=1)
    b1 = jnp.concatenate([b_p1, b_e1])[None, :]
    z64 = jnp.zeros((64, 64), jnp.float32)
    wn2 = jnp.block([[wn_p2, z64], [z64, wn_e2]])
    ws2 = jnp.block([[ws_p2, z64], [z64, ws_e2]])
    b2 = jnp.concatenate([b_p2, b_e2])[None, :]

    e = edge_weight.shape[0]
    src2 = src.reshape(e // CHUNK, CHUNK)
    dst2 = dst.reshape(e // CHUNK, CHUNK)
    ew2 = edge_weight.reshape(e // CHUNK, CHUNK)

    P = _seg_sum_sc(x, src2, dst2, ew2)
    H = _tc_layer1(P, x, wn1, ws1, b1)
    Q = _seg_sum_sc(H, src2, dst2, ew2)
    s1, LX = _tc_layer2(Q, H, wn2, ws2, b2)
    R = _seg_sum_sc(LX, dst2, src2, ew2)

    (g2wn1, g2ws1, g2b1), (g2wn2, g2ws2, g2b2) = params['gnn2']
    out, x1 = _tc_final(LX, R,
                        g2wn1, g2ws1, g2b1[None, :],
                        g2wn2, g2ws2, g2b2[None, :],
                        params['lin1_w'], params['lin1_b'][None, :],
                        params['lin2_w'], params['lin2_b'][None, :])
    return (out, s1, x1)


# speedup vs baseline: 1.3268x; 1.0473x over previous
import functools

import jax
import jax.numpy as jnp
from jax import lax
from jax.experimental import pallas as pl
from jax.experimental.pallas import tpu as pltpu
from jax.experimental.pallas import tpu_sc as plsc

NC = 2
NS = 16
NW = NC * NS
CHUNK = 128
NROW = 2
NIDX = 4


GSPLIT = 2


def _gather_split(table_h, idx_ref, rows_ref, sem):
    hw = CHUNK // GSPLIT
    for h in range(GSPLIT):
        pltpu.async_copy(table_h.at[idx_ref.at[pl.ds(h * hw, hw)]],
                         rows_ref.at[pl.ds(h * hw, hw)], sem)


def _gather_split_wait(table_h, idx_ref, rows_ref, sem):
    hw = CHUNK // GSPLIT
    for h in range(GSPLIT):
        pltpu.make_async_copy(table_h.at[idx_ref.at[pl.ds(h * hw, hw)]],
                              rows_ref.at[pl.ds(h * hw, hw)], sem).wait()


def _seg_sum_sc(table, g2, s2, ew2, tc_tiling=True):
    n, f = table.shape
    tot_chunks = g2.shape[0]
    nsl = f // 16
    rpt = (n // NS) // 8 * 8
    rem = n - rpt * NS
    nfull = rpt // CHUNK
    ztail = rpt - nfull * CHUNK
    assert rem % 8 == 0 and rem <= CHUNK and ztail % 8 == 0
    base_ck = (tot_chunks // NW) // NIDX * NIDX
    nhi = (tot_chunks - base_ck * NW) // NIDX
    wlo = NW - nhi
    assert base_ck * NW + nhi * NIDX == tot_chunks

    mesh = plsc.VectorSubcoreMesh(core_axis_name="c", subcore_axis_name="s")

    @functools.partial(
        pl.kernel,
        out_type=jax.ShapeDtypeStruct((NC, n, f), jnp.float32),
        mesh=mesh,
        compiler_params=pltpu.CompilerParams(use_tc_tiling_on_sc=tc_tiling),
        scratch_types=[
            [pltpu.VMEM((CHUNK,), jnp.int32)] * NIDX,
            [pltpu.VMEM((CHUNK,), jnp.int32)] * NIDX,
            [pltpu.VMEM((CHUNK,), jnp.float32)] * NIDX,
            [pltpu.VMEM((CHUNK, f), jnp.float32)] * NROW,
            pltpu.VMEM_SHARED((n, f), jnp.float32),
            [pltpu.SemaphoreType.DMA] * NIDX,
            [pltpu.SemaphoreType.DMA] * NROW,
            [pltpu.SemaphoreType.DMA] * NROW,
        ],
    )
    def ksc(table_h, g_h, s_h, ew_h, out_h, g_b, s_b, ew_b, rows, acc,
            isem, gsem, ssem):
        c_ax = lax.axis_index("c")
        s_ax = lax.axis_index("s")
        wid = c_ax * NS + s_ax
        nck = base_ck + jnp.where(wid >= wlo, NIDX, 0)
        cbase = wid * base_ck + jnp.maximum(0, wid - wlo) * NIDX
        zero16 = jnp.zeros((16,), jnp.float32)

        def idx_load(rel_c, st):
            gc = cbase + rel_c
            pltpu.async_copy(g_h.at[gc], g_b[st], isem[st])
            pltpu.async_copy(s_h.at[gc], s_b[st], isem[st])
            pltpu.async_copy(ew_h.at[gc], ew_b[st], isem[st])

        def idx_wait(rel_c, st):
            gc = cbase + rel_c
            pltpu.make_async_copy(g_h.at[gc], g_b[st], isem[st]).wait()
            pltpu.make_async_copy(s_h.at[gc], s_b[st], isem[st]).wait()
            pltpu.make_async_copy(ew_h.at[gc], ew_b[st], isem[st]).wait()

        for st in range(3):
            idx_load(st, st)

        def zrow(r, carry):
            for j in range(nsl):
                rows[0][r, pl.ds(j * 16, 16)] = zero16
            return carry

        lax.fori_loop(0, CHUNK, zrow, 0)
        row0 = pl.multiple_of(s_ax * rpt, 8)
        for zi in range(nfull):
            pltpu.sync_copy(rows[0], acc.at[pl.ds(row0 + zi * CHUNK, CHUNK)])
        if ztail:
            pltpu.sync_copy(rows[0].at[pl.ds(0, ztail)],
                            acc.at[pl.ds(row0 + nfull * CHUNK, ztail)])
        if rem:
            @pl.when(s_ax == 0)
            def _():
                pltpu.sync_copy(rows[0].at[pl.ds(0, rem)],
                                acc.at[pl.ds(NS * rpt, rem)])
        plsc.subcore_barrier()

        idx_wait(0, 0)
        _gather_split(table_h, g_b[0], rows[0], gsem[0])

        def phase(i, b):
            c = i * NIDX + b
            br = b % NROW
            b1 = (b + 1) % NROW
            st1 = (b + 1) % NIDX
            st3 = (b + 3) % NIDX
            _gather_split_wait(table_h, g_b[b], rows[br], gsem[br])

            @pl.when(c + 1 < nck)
            def _():
                @pl.when(c >= 1)
                def _():
                    pltpu.make_async_copy(rows[b1], acc.at[s_b[st3]],
                                          ssem[b1]).wait()

                @pl.when(c + 3 < nck)
                def _():
                    idx_load(c + 3, st3)
                idx_wait(c + 1, st1)
                _gather_split(table_h, g_b[st1], rows[b1], gsem[b1])

            for g in range(CHUNK // 16):
                wv = ew_b[b][pl.ds(g * 16, 16)]

                def lane_body(l, cc2, _wv=wv, _br=br, _g=g):
                    k = _g * 16 + l
                    w = lax.gather(
                        _wv, jnp.full((16, 1), l, jnp.int32),
                        lax.GatherDimensionNumbers(
                            offset_dims=(), collapsed_slice_dims=(0,),
                            start_index_map=(0,)),
                        slice_sizes=(1,),
                        mode=lax.GatherScatterMode.PROMISE_IN_BOUNDS)
                    for j in range(nsl):
                        rows[_br][k, pl.ds(j * 16, 16)] = (
                            rows[_br][k, pl.ds(j * 16, 16)] * w)
                    return cc2

                lax.fori_loop(0, 16, lane_body, 0, unroll=2)
            pltpu.async_copy(rows[br], acc.at[s_b[b]], ssem[br], add=True)

        def outer(i, carry):
            for b in range(NIDX):
                phase(i, b)
            return carry

        lax.fori_loop(0, nck // NIDX, outer, 0)

        for br in range(NROW):
            pltpu.make_async_copy(rows[br], acc.at[s_b[0]], ssem[br]).wait()

        plsc.subcore_barrier()
        pltpu.sync_copy(acc.at[pl.ds(row0, rpt)],
                        out_h.at[c_ax, pl.ds(row0, rpt)])
        if rem:
            @pl.when(s_ax == 0)
            def _():
                pltpu.sync_copy(acc.at[pl.ds(NS * rpt, rem)],
                                out_h.at[c_ax, pl.ds(NS * rpt, rem)])

    return ksc(table, g2, s2, ew2)


def _tc_layer1(P, x, wn, ws, b):
    n = x.shape[0]
    bn = 1000

    def body(p_ref, x_ref, wn_ref, ws_ref, b_ref, o_ref):
        agg = p_ref[0] + p_ref[1]
        h = (jnp.dot(agg, wn_ref[...], preferred_element_type=jnp.float32)
             + jnp.dot(x_ref[...], ws_ref[...], preferred_element_type=jnp.float32)
             + b_ref[...])
        o_ref[...] = jnp.maximum(h, 0.0)

    return pl.pallas_call(
        body,
        grid=(n // bn,),
        in_specs=[
            pl.BlockSpec((2, bn, 128), lambda i: (0, i, 0)),
            pl.BlockSpec((bn, 128), lambda i: (i, 0)),
            pl.BlockSpec((128, 128), lambda i: (0, 0)),
            pl.BlockSpec((128, 128), lambda i: (0, 0)),
            pl.BlockSpec((1, 128), lambda i: (0, 0)),
        ],
        out_specs=pl.BlockSpec((bn, 128), lambda i: (i, 0)),
        out_shape=jax.ShapeDtypeStruct((n, 128), jnp.float32),
    )(P, x, wn, ws, b)


def _tc_layer2(Q, H, wn, ws, b):
    n = H.shape[0]
    bn = 1000

    def body(q_ref, h_ref, wn_ref, ws_ref, b_ref, s_ref, ls_ref, xe_ref):
        t = (jnp.dot(q_ref[0] + q_ref[1], wn_ref[...], preferred_element_type=jnp.float32)
             + jnp.dot(h_ref[...], ws_ref[...], preferred_element_type=jnp.float32)
             + b_ref[...])
        sblk = t[:, :64]
        s_ref[...] = sblk
        m = jnp.max(sblk, axis=1, keepdims=True)
        sh = sblk - m
        ls_ref[...] = sh - jnp.log(jnp.sum(jnp.exp(sh), axis=1, keepdims=True))
        xe_ref[...] = t[:, 64:]

    return pl.pallas_call(
        body,
        grid=(n // bn,),
        in_specs=[
            pl.BlockSpec((2, bn, 128), lambda i: (0, i, 0)),
            pl.BlockSpec((bn, 128), lambda i: (i, 0)),
            pl.BlockSpec((128, 128), lambda i: (0, 0)),
            pl.BlockSpec((128, 128), lambda i: (0, 0)),
            pl.BlockSpec((1, 128), lambda i: (0, 0)),
        ],
        out_specs=[
            pl.BlockSpec((bn, 64), lambda i: (i, 0)),
            pl.BlockSpec((bn, 64), lambda i: (i, 0)),
            pl.BlockSpec((bn, 64), lambda i: (i, 0)),
        ],
        out_shape=[
            jax.ShapeDtypeStruct((n, 64), jnp.float32),
            jax.ShapeDtypeStruct((n, 64), jnp.float32),
            jax.ShapeDtypeStruct((n, 64), jnp.float32),
        ],
    )(Q, H, wn, ws, b)


def _tc_final(S, xe, R, wn1, ws1, b1, wn2, ws2, b2, l1w, l1b, l2w, l2b):
    n = S.shape[0]
    bn = 1000
    nb = n // bn
    cdim = (((0,), (0,)), ((), ()))

    def body(s_ref, xe_ref, r_ref, wn1_ref, ws1_ref, b1_ref, wn2_ref, ws2_ref,
             b2_ref, l1w_ref, l1b_ref, l2w_ref, l2b_ref, out_ref, x1_ref,
             a2_acc, xp_acc):
        i = pl.program_id(0)

        @pl.when(i == 0)
        def _():
            a2_acc[...] = jnp.zeros_like(a2_acc)
            xp_acc[...] = jnp.zeros_like(xp_acc)

        sb = s_ref[...]
        asb = r_ref[0] + r_ref[1]
        a2_acc[...] += lax.dot_general(sb, asb, cdim,
                                       preferred_element_type=jnp.float32)
        xp_acc[...] += lax.dot_general(sb, xe_ref[...], cdim,
                                       preferred_element_type=jnp.float32)

        @pl.when(i == nb - 1)
        def _():
            A2 = a2_acc[...]
            xp = xp_acc[...]
            x1_ref[...] = xp
            h = jnp.maximum(
                jnp.dot(jnp.dot(A2, xp, preferred_element_type=jnp.float32),
                        wn1_ref[...], preferred_element_type=jnp.float32)
                + jnp.dot(xp, ws1_ref[...], preferred_element_type=jnp.float32)
                + b1_ref[...], 0.0)
            h = (jnp.dot(jnp.dot(A2, h, preferred_element_type=jnp.float32),
                         wn2_ref[...], preferred_element_type=jnp.float32)
                 + jnp.dot(h, ws2_ref[...], preferred_element_type=jnp.float32)
                 + b2_ref[...])
            xm = jnp.mean(h, axis=0, keepdims=True)
            z = jnp.dot(xm, l1w_ref[...], preferred_element_type=jnp.float32) + l1b_ref[...]
            z = jnp.where(z >= 0, z, 0.01 * z)
            z = jnp.dot(z, l2w_ref[...], preferred_element_type=jnp.float32) + l2b_ref[...]
            m = jnp.max(z, axis=1, keepdims=True)
            zs = z - m
            out_ref[...] = zs - jnp.log(jnp.sum(jnp.exp(zs), axis=1, keepdims=True))

    full = lambda shape: pl.BlockSpec(shape, lambda i: tuple(0 for _ in shape))
    return pl.pallas_call(
        body,
        grid=(nb,),
        in_specs=[
            pl.BlockSpec((bn, 64), lambda i: (i, 0)),
            pl.BlockSpec((bn, 64), lambda i: (i, 0)),
            pl.BlockSpec((2, bn, 64), lambda i: (0, i, 0)),
            full((64, 64)), full((64, 64)), full((1, 64)),
            full((64, 64)), full((64, 64)), full((1, 64)),
            full((64, 32)), full((1, 32)), full((32, 32)), full((1, 32)),
        ],
        out_specs=[
            pl.BlockSpec((1, 32), lambda i: (0, 0)),
            pl.BlockSpec((64, 64), lambda i: (0, 0)),
        ],
        out_shape=[
            jax.ShapeDtypeStruct((1, 32), jnp.float32),
            jax.ShapeDtypeStruct((64, 64), jnp.float32),
        ],
        scratch_shapes=[
            pltpu.VMEM((64, 64), jnp.float32),
            pltpu.VMEM((64, 64), jnp.float32),
        ],
    )(S, xe, R, wn1, ws1, b1, wn2, ws2, b2, l1w, l1b, l2w, l2b)


def kernel(x, edge_index, edge_weight, params):
    src = edge_index[0]
    dst = edge_index[1]
    (wn_p1, ws_p1, b_p1), (wn_p2, ws_p2, b_p2) = params['pool']
    (wn_e1, ws_e1, b_e1), (wn_e2, ws_e2, b_e2) = params['embed']
    wn1 = jnp.concatenate([wn_p1, wn_e1], axis=1)
    ws1 = jnp.concatenate([ws_p1, ws_e1], axis=1)
    b1 = jnp.concatenate([b_p1, b_e1])[None, :]
    z64 = jnp.zeros((64, 64), jnp.float32)
    wn2 = jnp.block([[wn_p2, z64], [z64, wn_e2]])
    ws2 = jnp.block([[ws_p2, z64], [z64, ws_e2]])
    b2 = jnp.concatenate([b_p2, b_e2])[None, :]

    e = edge_weight.shape[0]
    src2 = src.reshape(e // CHUNK, CHUNK)
    dst2 = dst.reshape(e // CHUNK, CHUNK)
    ew2 = edge_weight.reshape(e // CHUNK, CHUNK)

    P = _seg_sum_sc(x, src2, dst2, ew2)
    H = _tc_layer1(P, x, wn1, ws1, b1)
    Q = _seg_sum_sc(H, src2, dst2, ew2)
    s1, S, xe = _tc_layer2(Q, H, wn2, ws2, b2)
    R = _seg_sum_sc(S, dst2, src2, ew2, tc_tiling=False)

    (g2wn1, g2ws1, g2b1), (g2wn2, g2ws2, g2b2) = params['gnn2']
    out, x1 = _tc_final(S, xe, R,
                        g2wn1, g2ws1, g2b1[None, :],
                        g2wn2, g2ws2, g2b2[None, :],
                        params['lin1_w'], params['lin1_b'][None, :],
                        params['lin2_w'], params['lin2_b'][None, :])
    return (out, s1, x1)
